# baseline (speedup 1.0000x reference)
import functools

import jax
import jax.numpy as jnp
from jax import lax
from jax.experimental import pallas as pl
from jax.experimental.pallas import tpu as pltpu
from jax.experimental.pallas import tpu_sc as plsc

_NC = 2
_NS = 16
_NW = _NC * _NS
_CH = 128


@functools.lru_cache(maxsize=None)
def _make_gather(B, V, D):
    assert B % _NW == 0
    b_per_w = B // _NW
    assert b_per_w % _CH == 0
    nch = b_per_w // _CH

    mesh = plsc.VectorSubcoreMesh(core_axis_name="c", subcore_axis_name="s")

    @functools.partial(
        pl.kernel,
        out_type=jax.ShapeDtypeStruct((B, D), jnp.float32),
        mesh=mesh,
        scratch_types=[
            pltpu.VMEM((b_per_w,), jnp.int32),
            pltpu.VMEM((_CH, D), jnp.float32),
            pltpu.SemaphoreType.DMA,
        ],
        compiler_params=pltpu.CompilerParams(use_tc_tiling_on_sc=False),
    )
    def k(idx_hbm, table_hbm, out_hbm, idx_v, rows_v, sem):
        wid = lax.axis_index("s") * _NC + lax.axis_index("c")
        base = wid * b_per_w
        pltpu.sync_copy(idx_hbm.at[pl.ds(base, b_per_w)], idx_v)

        @pl.loop(0, nch)
        def _chunk(c):
            off = c * _CH
            pltpu.async_copy(
                table_hbm.at[idx_v.at[pl.ds(off, _CH)]], rows_v, sem
            ).wait()
            pltpu.sync_copy(rows_v, out_hbm.at[pl.ds(base + off, _CH)])

    return k


def kernel(tokens, wte):
    bsz, seq = tokens.shape
    v, d = wte.shape
    idx = tokens.reshape(-1).astype(jnp.int32)
    out = _make_gather(idx.shape[0], v, d)(idx, wte)
    return out.reshape(bsz, seq, d)


# speedup vs baseline: 1.1185x; 1.1185x over previous
import functools

import jax
import jax.numpy as jnp
from jax import lax
from jax.experimental import pallas as pl
from jax.experimental.pallas import tpu as pltpu
from jax.experimental.pallas import tpu_sc as plsc

_NC = 2
_NS = 16
_NW = _NC * _NS
_CH = 128
_K = 8
_G = 4


@functools.lru_cache(maxsize=None)
def _make_gather(B, V, D):
    assert B % _NW == 0
    b_per_w = B // _NW
    assert b_per_w % _CH == 0
    nch = b_per_w // _CH
    assert nch % _K == 0 and nch >= 2 * _K

    mesh = plsc.VectorSubcoreMesh(core_axis_name="c", subcore_axis_name="s")

    @functools.partial(
        pl.kernel,
        out_type=jax.ShapeDtypeStruct((B, D), jnp.float32),
        mesh=mesh,
        scratch_types=[
            pltpu.VMEM((b_per_w,), jnp.int32),
            pltpu.VMEM((_K, _CH, D), jnp.float32),
            pltpu.SemaphoreType.DMA((_K,)),
            pltpu.SemaphoreType.DMA((_K,)),
        ],
        compiler_params=pltpu.CompilerParams(use_tc_tiling_on_sc=False),
    )
    def k(idx_hbm, table_hbm, out_hbm, idx_v, rows_v, gsem, ssem):
        wid = lax.axis_index("s") * _NC + lax.axis_index("c")
        base = wid * b_per_w
        pltpu.sync_copy(idx_hbm.at[pl.ds(base, b_per_w)], idx_v)

        def start_gather(c, b):
            pltpu.async_copy(
                table_hbm.at[idx_v.at[pl.ds(c * _CH, _CH)]],
                rows_v.at[b],
                gsem.at[b],
            )

        def wait_gather(c, b):
            pltpu.make_async_copy(
                table_hbm.at[idx_v.at[pl.ds(c * _CH, _CH)]],
                rows_v.at[b],
                gsem.at[b],
            ).wait()

        def start_store(c, b):
            pltpu.async_copy(
                rows_v.at[b], out_hbm.at[pl.ds(base + c * _CH, _CH)], ssem.at[b]
            )

        def wait_store(c, b):
            pltpu.make_async_copy(
                rows_v.at[b], out_hbm.at[pl.ds(base + c * _CH, _CH)], ssem.at[b]
            ).wait()

        for b in range(_G):
            start_gather(b, b)

        for c in range(_K):
            wait_gather(c, c)
            start_store(c, c)
            if c >= _K - _G:
                wait_store(c - (_K - _G), (c + _G) % _K)
            start_gather(c + _G, (c + _G) % _K)

        @pl.loop(_K, nch - _K, step=_K)
        def _super(c0):
            for b in range(_K):
                c = c0 + b
                wait_gather(c, b)
                start_store(c, b)
                g = (b + _G) % _K
                wait_store(c - (_K - _G), g)
                start_gather(c + _G, g)

        for b in range(_K):
            c = nch - _K + b
            wait_gather(c, b)
            start_store(c, b)
            if b < _K - _G:
                g = (b + _G) % _K
                wait_store(c - (_K - _G), g)
                start_gather(c + _G, g)
        for b in range(_K):
            wait_store(nch - _K + b, b)

    return k


def kernel(tokens, wte):
    bsz, seq = tokens.shape
    v, d = wte.shape
    idx = tokens.reshape(-1).astype(jnp.int32)
    out = _make_gather(idx.shape[0], v, d)(idx, wte)
    return out.reshape(bsz, seq, d)
